# Initial kernel scaffold; baseline (speedup 1.0000x reference)
import jax
import jax.numpy as jnp
from jax.experimental import pallas as pl


def kernel(edge_index, user_features, book_num_features, book_genre_features, emb_table, W_user, b_user, W_bnum, b_bnum, W_bgen, b_bgen):
    raise NotImplementedError("write your pallas kernel here")


# speedup vs baseline: 5.9233x; 5.9233x over previous
import functools

import jax
import jax.numpy as jnp
from jax import lax
from jax.experimental import pallas as pl
from jax.experimental.pallas import tpu as pltpu
from jax.experimental.pallas import tpu_sc as plsc

N_USERS = 4000
N_NODES = 10000
N_EDGES = 320000
D = 128

NC = 2
NS = 16
NW = NC * NS

CB = 64
NCHUNK = 80
NPASS = 2
EW = CB * NCHUNK * NPASS
EPAD = EW * NW

ROWS = 10240
RPS = ROWS // NS
DEGW = 16


def _mesh():
    return plsc.VectorSubcoreMesh(core_axis_name="c", subcore_axis_name="s")


def _sc_deg(dstw, zD):
    @functools.partial(
        pl.kernel,
        mesh=_mesh(),
        out_type=jax.ShapeDtypeStruct((NC, ROWS, D), jnp.float32),
        scratch_types=[
            pltpu.VMEM((NCHUNK, CB), jnp.int32),
            pltpu.VMEM((CB, D), jnp.float32),
            pltpu.VMEM_SHARED((ROWS, D), jnp.float32),
        ],
    )
    def body(dstw_hbm, zD_hbm, out_hbm, dst_v, ones_v, acc_sh):
        cid = lax.axis_index("c")
        sid = lax.axis_index("s")
        wid = sid * NC + cid

        @pl.when(sid == 0)
        def _():
            pltpu.sync_copy(zD_hbm, acc_sh)

        orow = jnp.ones((16,), jnp.float32)

        def _ones(i, carry):
            for j in range(D // 16):
                ones_v[i, pl.ds(j * 16, 16)] = orow
            return carry

        lax.fori_loop(0, CB, _ones, 0)
        plsc.subcore_barrier()

        for p in range(NPASS):
            pltpu.sync_copy(dstw_hbm.at[wid, p], dst_v)

            def _scat(j, carry):
                pltpu.sync_copy(ones_v, acc_sh.at[dst_v.at[j]], add=True)
                return carry

            lax.fori_loop(0, NCHUNK, _scat, 0)

        plsc.subcore_barrier()

        @pl.when(sid == 0)
        def _():
            pltpu.sync_copy(acc_sh, out_hbm.at[cid])

    return body(dstw, zD)


def _sc_layer(embS, srcw, dstw, zD):
    @functools.partial(
        pl.kernel,
        mesh=_mesh(),
        out_type=jax.ShapeDtypeStruct((NC, ROWS, D), jnp.float32),
        scratch_types=[
            pltpu.VMEM((NCHUNK, CB), jnp.int32),
            pltpu.VMEM((NCHUNK, CB), jnp.int32),
            pltpu.VMEM((CB, D), jnp.float32),
            pltpu.VMEM((CB, D), jnp.float32),
            pltpu.VMEM_SHARED((ROWS, D), jnp.float32),
            pltpu.SemaphoreType.DMA,
            pltpu.SemaphoreType.DMA,
        ],
    )
    def body(embS_hbm, srcw_hbm, dstw_hbm, zD_hbm, out_hbm,
             src_v, dst_v, buf0, buf1, acc_sh, sem0, sem1):
        cid = lax.axis_index("c")
        sid = lax.axis_index("s")
        wid = sid * NC + cid

        @pl.when(sid == 0)
        def _():
            pltpu.sync_copy(zD_hbm, acc_sh)

        plsc.subcore_barrier()

        bufs = (buf0, buf1)
        sems = (sem0, sem1)
        for p in range(NPASS):
            pltpu.sync_copy(srcw_hbm.at[wid, p], src_v)
            pltpu.sync_copy(dstw_hbm.at[wid, p], dst_v)
            pltpu.async_copy(embS_hbm.at[src_v.at[0]], buf0, sem0)
            pltpu.async_copy(embS_hbm.at[src_v.at[1]], buf1, sem1)

            def _step(i, carry):
                g = i * 2
                for b in range(2):
                    j = g + b
                    pltpu.make_async_copy(embS_hbm.at[src_v.at[j]], bufs[b], sems[b]).wait()
                    pltpu.sync_copy(bufs[b], acc_sh.at[dst_v.at[j]], add=True)
                    pltpu.async_copy(embS_hbm.at[src_v.at[j + 2]], bufs[b], sems[b])
                return carry

            lax.fori_loop(0, (NCHUNK - 2) // 2, _step, 0)
            for b in range(2):
                j = NCHUNK - 2 + b
                pltpu.make_async_copy(embS_hbm.at[src_v.at[j]], bufs[b], sems[b]).wait()
                pltpu.sync_copy(bufs[b], acc_sh.at[dst_v.at[j]], add=True)

        plsc.subcore_barrier()

        @pl.when(sid == 0)
        def _():
            pltpu.sync_copy(acc_sh, out_hbm.at[cid])

    return body(embS, srcw, dstw, zD)


def _tc_proj(uf, bn, bg, et, wu, bu, wn, bb, wg, bg2):
    def body(uf_r, bn_r, bg_r, et_r, wu_r, bu_r, wn_r, bb_r, wg_r, bg2_r, emb0_o):
        dn = (((1,), (1,)), ((), ()))
        hi = lax.Precision.HIGHEST
        up = lax.dot_general(uf_r[...], wu_r[...], dn, precision=hi,
                             preferred_element_type=jnp.float32) + bu_r[...]
        bp = (lax.dot_general(bn_r[...], wn_r[...], dn, precision=hi,
                              preferred_element_type=jnp.float32) + bb_r[...]
              + lax.dot_general(bg_r[...], wg_r[...], dn, precision=hi,
                                preferred_element_type=jnp.float32) + bg2_r[...])
        emb0_o[0:N_USERS, :] = et_r[0:N_USERS, :] + up
        emb0_o[N_USERS:N_NODES, :] = et_r[N_USERS:N_NODES, :] + bp

    return pl.pallas_call(
        body,
        out_shape=jax.ShapeDtypeStruct((N_NODES, D), jnp.float32),
    )(uf, bn, bg, et, wu, bu, wn, bb, wg, bg2)


def _tc_scale(degp, emb0p):
    def body(degp_r, emb0p_r, dinvB_o, embS_o):
        deg = degp_r[0, :, 0:1] + degp_r[1, :, 0:1]
        dinv = jnp.where(deg > 0.0, lax.rsqrt(deg), 0.0)
        dinvB = jnp.broadcast_to(dinv, (ROWS, D))
        dinvB_o[...] = dinvB
        embS_o[...] = emb0p_r[...] * dinvB

    return pl.pallas_call(
        body,
        out_shape=[
            jax.ShapeDtypeStruct((ROWS, D), jnp.float32),
            jax.ShapeDtypeStruct((ROWS, D), jnp.float32),
        ],
    )(degp, emb0p)


def _tc_combine(p, dinvB, accp, final):
    def body(p_r, dinv_r, acc_r, *outs):
        s = (p_r[0] + p_r[1]) * dinv_r[...]
        a = acc_r[...] + s
        if final:
            outs[0][...] = a[0:N_NODES, :] * 0.25
        else:
            outs[0][...] = s * dinv_r[...]
            outs[1][...] = a

    if final:
        shapes = [jax.ShapeDtypeStruct((N_NODES, D), jnp.float32)]
    else:
        shapes = [jax.ShapeDtypeStruct((ROWS, D), jnp.float32),
                  jax.ShapeDtypeStruct((ROWS, D), jnp.float32)]
    return pl.pallas_call(body, out_shape=shapes)(p, dinvB, accp)


def kernel(edge_index, user_features, book_num_features, book_genre_features,
           emb_table, W_user, b_user, W_bnum, b_bnum, W_bgen, b_bgen):
    src = edge_index[0].astype(jnp.int32)
    dst = edge_index[1].astype(jnp.int32)
    npad = EPAD - N_EDGES
    srcw = jnp.concatenate([src, jnp.zeros((npad,), jnp.int32)]).reshape(NW, NPASS, NCHUNK, CB)
    dstw = jnp.concatenate([dst, jnp.full((npad,), N_NODES, jnp.int32)]).reshape(NW, NPASS, NCHUNK, CB)

    zD = jnp.zeros((ROWS, D), jnp.float32)
    degp = _sc_deg(dstw, zD)
    emb0 = _tc_proj(
        user_features, book_num_features, book_genre_features, emb_table,
        W_user, b_user.reshape(1, D), W_bnum, b_bnum.reshape(1, D),
        W_bgen, b_bgen.reshape(1, D))
    emb0p = jnp.pad(emb0, ((0, ROWS - N_NODES), (0, 0)))
    dinvB, embS = _tc_scale(degp, emb0p)

    acc = emb0p
    out = None
    for layer in range(3):
        p = _sc_layer(embS, srcw, dstw, zD)
        if layer < 2:
            embS, acc = _tc_combine(p, dinvB, acc, final=False)
        else:
            out = _tc_combine(p, dinvB, acc, final=True)[0]
    return emb0, out


# speedup vs baseline: 6.5985x; 1.1140x over previous
import functools

import jax
import jax.numpy as jnp
from jax import lax
from jax.experimental import pallas as pl
from jax.experimental.pallas import tpu as pltpu
from jax.experimental.pallas import tpu_sc as plsc

N_USERS = 4000
N_NODES = 10000
N_EDGES = 320000
D = 128

NC = 2
NS = 16
NW = NC * NS

CB = 64
NCHUNK = 80
NPASS = 2
EW = CB * NCHUNK * NPASS
EPAD = EW * NW

C_FAST = 118
C_SLOW = 42
FAST_CORE = 0

ROWS = 10240
RPS = ROWS // NS
DEGW = 16


def _mesh():
    return plsc.VectorSubcoreMesh(core_axis_name="c", subcore_axis_name="s")


def _sc_deg(dstw, zD):
    @functools.partial(
        pl.kernel,
        mesh=_mesh(),
        out_type=jax.ShapeDtypeStruct((NC, ROWS, D), jnp.float32),
        scratch_types=[
            pltpu.VMEM((NCHUNK, CB), jnp.int32),
            pltpu.VMEM((CB, D), jnp.float32),
            pltpu.VMEM_SHARED((ROWS, D), jnp.float32),
        ],
    )
    def body(dstw_hbm, zD_hbm, out_hbm, dst_v, ones_v, acc_sh):
        cid = lax.axis_index("c")
        sid = lax.axis_index("s")
        wid = sid * NC + cid

        @pl.when(sid == 0)
        def _():
            pltpu.sync_copy(zD_hbm, acc_sh)

        orow = jnp.ones((16,), jnp.float32)

        def _ones(i, carry):
            for j in range(D // 16):
                ones_v[i, pl.ds(j * 16, 16)] = orow
            return carry

        lax.fori_loop(0, CB, _ones, 0)
        plsc.subcore_barrier()

        for p in range(NPASS):
            pltpu.sync_copy(dstw_hbm.at[wid, p], dst_v)

            def _scat(j, carry):
                pltpu.sync_copy(ones_v, acc_sh.at[dst_v.at[j]], add=True)
                return carry

            lax.fori_loop(0, NCHUNK, _scat, 0)

        plsc.subcore_barrier()

        @pl.when(sid == 0)
        def _():
            pltpu.sync_copy(acc_sh, out_hbm.at[cid])

    return body(dstw, zD)


def _sc_layer(embS, srcf, dstf, srcs, dsts, zD):
    @functools.partial(
        pl.kernel,
        mesh=_mesh(),
        out_type=jax.ShapeDtypeStruct((NC, ROWS, D), jnp.float32),
        scratch_types=[
            pltpu.VMEM((C_FAST, CB), jnp.int32),
            pltpu.VMEM((C_FAST, CB), jnp.int32),
            pltpu.VMEM((CB, D), jnp.float32),
            pltpu.VMEM((CB, D), jnp.float32),
            pltpu.VMEM_SHARED((ROWS, D), jnp.float32),
            pltpu.SemaphoreType.DMA,
            pltpu.SemaphoreType.DMA,
        ],
    )
    def body(embS_hbm, srcf_hbm, dstf_hbm, srcs_hbm, dsts_hbm, zD_hbm, out_hbm,
             src_v, dst_v, buf0, buf1, acc_sh, sem0, sem1):
        cid = lax.axis_index("c")
        sid = lax.axis_index("s")

        @pl.when(sid == 0)
        def _():
            pltpu.sync_copy(zD_hbm, acc_sh)

        plsc.subcore_barrier()

        bufs = (buf0, buf1)
        sems = (sem0, sem1)

        def edge_phase(src_hbm, dst_hbm, nchunk):
            for p in range(NPASS):
                pltpu.sync_copy(src_hbm.at[sid, p], src_v.at[pl.ds(0, nchunk)])
                pltpu.sync_copy(dst_hbm.at[sid, p], dst_v.at[pl.ds(0, nchunk)])
                pltpu.async_copy(embS_hbm.at[src_v.at[0]], buf0, sem0)
                pltpu.async_copy(embS_hbm.at[src_v.at[1]], buf1, sem1)

                def _step(i, carry):
                    g = i * 2
                    for b in range(2):
                        j = g + b
                        pltpu.make_async_copy(embS_hbm.at[src_v.at[j]], bufs[b], sems[b]).wait()
                        pltpu.sync_copy(bufs[b], acc_sh.at[dst_v.at[j]], add=True)
                        pltpu.async_copy(embS_hbm.at[src_v.at[j + 2]], bufs[b], sems[b])
                    return carry

                lax.fori_loop(0, (nchunk - 2) // 2, _step, 0)
                for b in range(2):
                    j = nchunk - 2 + b
                    pltpu.make_async_copy(embS_hbm.at[src_v.at[j]], bufs[b], sems[b]).wait()
                    pltpu.sync_copy(bufs[b], acc_sh.at[dst_v.at[j]], add=True)

        @pl.when(cid == FAST_CORE)
        def _():
            edge_phase(srcf_hbm, dstf_hbm, C_FAST)

        @pl.when(cid == 1 - FAST_CORE)
        def _():
            edge_phase(srcs_hbm, dsts_hbm, C_SLOW)

        plsc.subcore_barrier()

        @pl.when(sid == 0)
        def _():
            pltpu.sync_copy(acc_sh, out_hbm.at[cid])

    return body(embS, srcf, dstf, srcs, dsts, zD)


def _tc_proj(uf, bn, bg, et, wu, bu, wn, bb, wg, bg2):
    def body(uf_r, bn_r, bg_r, et_r, wu_r, bu_r, wn_r, bb_r, wg_r, bg2_r, emb0_o):
        dn = (((1,), (1,)), ((), ()))
        hi = lax.Precision.HIGHEST
        up = lax.dot_general(uf_r[...], wu_r[...], dn, precision=hi,
                             preferred_element_type=jnp.float32) + bu_r[...]
        bp = (lax.dot_general(bn_r[...], wn_r[...], dn, precision=hi,
                              preferred_element_type=jnp.float32) + bb_r[...]
              + lax.dot_general(bg_r[...], wg_r[...], dn, precision=hi,
                                preferred_element_type=jnp.float32) + bg2_r[...])
        emb0_o[0:N_USERS, :] = et_r[0:N_USERS, :] + up
        emb0_o[N_USERS:N_NODES, :] = et_r[N_USERS:N_NODES, :] + bp

    return pl.pallas_call(
        body,
        out_shape=jax.ShapeDtypeStruct((N_NODES, D), jnp.float32),
    )(uf, bn, bg, et, wu, bu, wn, bb, wg, bg2)


def _tc_scale(degp, emb0p):
    def body(degp_r, emb0p_r, dinvB_o, embS_o):
        deg = degp_r[0, :, 0:1] + degp_r[1, :, 0:1]
        dinv = jnp.where(deg > 0.0, lax.rsqrt(deg), 0.0)
        dinvB = jnp.broadcast_to(dinv, (ROWS, D))
        dinvB_o[...] = dinvB
        embS_o[...] = emb0p_r[...] * dinvB

    return pl.pallas_call(
        body,
        out_shape=[
            jax.ShapeDtypeStruct((ROWS, D), jnp.float32),
            jax.ShapeDtypeStruct((ROWS, D), jnp.float32),
        ],
    )(degp, emb0p)


def _tc_combine(p, dinvB, accp, final):
    def body(p_r, dinv_r, acc_r, *outs):
        s = (p_r[0] + p_r[1]) * dinv_r[...]
        a = acc_r[...] + s
        if final:
            outs[0][...] = a[0:N_NODES, :] * 0.25
        else:
            outs[0][...] = s * dinv_r[...]
            outs[1][...] = a

    if final:
        shapes = [jax.ShapeDtypeStruct((N_NODES, D), jnp.float32)]
    else:
        shapes = [jax.ShapeDtypeStruct((ROWS, D), jnp.float32),
                  jax.ShapeDtypeStruct((ROWS, D), jnp.float32)]
    return pl.pallas_call(body, out_shape=shapes)(p, dinvB, accp)


def kernel(edge_index, user_features, book_num_features, book_genre_features,
           emb_table, W_user, b_user, W_bnum, b_bnum, W_bgen, b_bgen):
    src = edge_index[0].astype(jnp.int32)
    dst = edge_index[1].astype(jnp.int32)
    npad = EPAD - N_EDGES
    src_flat = jnp.concatenate([src, jnp.zeros((npad,), jnp.int32)])
    dst_flat = jnp.concatenate([dst, jnp.full((npad,), N_NODES, jnp.int32)])
    dstw = dst_flat.reshape(NW, NPASS, NCHUNK, CB)
    e_fast = NS * NPASS * C_FAST * CB
    srcw_f = src_flat[:e_fast].reshape(NS, NPASS, C_FAST, CB)
    dstw_f = dst_flat[:e_fast].reshape(NS, NPASS, C_FAST, CB)
    srcw_s = src_flat[e_fast:].reshape(NS, NPASS, C_SLOW, CB)
    dstw_s = dst_flat[e_fast:].reshape(NS, NPASS, C_SLOW, CB)

    zD = jnp.zeros((ROWS, D), jnp.float32)
    degp = _sc_deg(dstw, zD)
    emb0 = _tc_proj(
        user_features, book_num_features, book_genre_features, emb_table,
        W_user, b_user.reshape(1, D), W_bnum, b_bnum.reshape(1, D),
        W_bgen, b_bgen.reshape(1, D))
    emb0p = jnp.pad(emb0, ((0, ROWS - N_NODES), (0, 0)))
    dinvB, embS = _tc_scale(degp, emb0p)

    acc = emb0p
    out = None
    for layer in range(3):
        p = _sc_layer(embS, srcw_f, dstw_f, srcw_s, dstw_s, zD)
        if layer < 2:
            embS, acc = _tc_combine(p, dinvB, acc, final=False)
        else:
            out = _tc_combine(p, dinvB, acc, final=True)[0]
    return emb0, out
